# baseline (speedup 1.0000x reference)
import functools

import jax
import jax.numpy as jnp
from jax.experimental import pallas as pl
from jax.experimental.pallas import tpu as pltpu

F32 = jnp.float32


def _embed_body(x_ref, w_ref, b_ref, g_ref, bb_ref, pe_ref, o_ref):
    h = jnp.dot(x_ref[0], w_ref[...], preferred_element_type=F32) + b_ref[...]
    m = h.mean(-1, keepdims=True)
    v = ((h - m) ** 2).mean(-1, keepdims=True)
    h = (h - m) / jnp.sqrt(v + 1e-5) * g_ref[...] + bb_ref[...]
    o_ref[...] = h + pe_ref[...]


def _embed(x, w, b, g, bb, pe):
    G, S, D_IN = x.shape
    DM = w.shape[1]
    return pl.pallas_call(
        _embed_body,
        grid=(G,),
        in_specs=[
            pl.BlockSpec((1, S, D_IN), lambda i: (i, 0, 0)),
            pl.BlockSpec((D_IN, DM), lambda i: (0, 0)),
            pl.BlockSpec((1, DM), lambda i: (0, 0)),
            pl.BlockSpec((1, DM), lambda i: (0, 0)),
            pl.BlockSpec((1, DM), lambda i: (0, 0)),
            pl.BlockSpec((S, DM), lambda i: (0, 0)),
        ],
        out_specs=pl.BlockSpec((S, DM), lambda i: (i, 0)),
        out_shape=jax.ShapeDtypeStruct((G * S, DM), F32),
    )(x, w, b, g, bb, pe)


def _proj_body(h_ref, wl_ref, bl_ref, wr_ref, br_ref, xl_ref, xr_ref):
    hb = h_ref[...]
    xl_ref[...] = jnp.dot(hb, wl_ref[...], preferred_element_type=F32) + bl_ref[...]
    xr_ref[...] = jnp.dot(hb, wr_ref[...], preferred_element_type=F32) + br_ref[...]


def _proj(h, wl, bl, wr, br, blk=512):
    N, DM = h.shape
    HC = wl.shape[1]
    return pl.pallas_call(
        _proj_body,
        grid=(N // blk,),
        in_specs=[
            pl.BlockSpec((blk, DM), lambda i: (i, 0)),
            pl.BlockSpec((DM, HC), lambda i: (0, 0)),
            pl.BlockSpec((1, HC), lambda i: (0, 0)),
            pl.BlockSpec((DM, HC), lambda i: (0, 0)),
            pl.BlockSpec((1, HC), lambda i: (0, 0)),
        ],
        out_specs=[
            pl.BlockSpec((blk, HC), lambda i: (i, 0)),
            pl.BlockSpec((blk, HC), lambda i: (i, 0)),
        ],
        out_shape=[
            jax.ShapeDtypeStruct((N, HC), F32),
            jax.ShapeDtypeStruct((N, HC), F32),
        ],
    )(h, wl, bl, wr, br)


def _eproj_body(ea_ref, we_ref, o_ref):
    o_ref[...] = jnp.dot(ea_ref[...], we_ref[...], preferred_element_type=F32)


def _eproj(ea, we, blk=2048):
    E, ED = ea.shape
    HC = we.shape[1]
    return pl.pallas_call(
        _eproj_body,
        grid=(E // blk,),
        in_specs=[
            pl.BlockSpec((blk, ED), lambda i: (i, 0)),
            pl.BlockSpec((ED, HC), lambda i: (0, 0)),
        ],
        out_specs=pl.BlockSpec((blk, HC), lambda i: (i, 0)),
        out_shape=jax.ShapeDtypeStruct((E, HC), F32),
    )(ea, we)


def _alpha_body(src_ref, dst_ref, xl_ref, xr_ref, el_ref, att_ref, o_ref,
                matt_ref, *, blk):
    att = att_ref[...]

    def body(j, carry):
        s = src_ref[0, 0, j]
        d = dst_ref[0, 0, j]
        z = xl_ref[s] + xr_ref[d] + el_ref[j]
        z = jnp.maximum(z, 0.2 * z) * att
        matt_ref[j] = z
        return carry

    jax.lax.fori_loop(0, blk, body, 0, unroll=4)
    msum = jnp.sum(matt_ref[...], axis=2)
    p = msum + pltpu.roll(msum, 7, 1)
    lane = jax.lax.broadcasted_iota(jnp.int32, msum.shape, 1)
    a8 = jnp.where(lane % 2 == 0, p, pltpu.roll(p, 1, 1))
    o_ref[...] = jnp.exp(a8)


def _alpha(src3, dst3, xlT, xrT, el8, att8, blk=1024):
    N = xlT.shape[0]
    E = el8.shape[0]
    return pl.pallas_call(
        functools.partial(_alpha_body, blk=blk),
        grid=(E // blk,),
        in_specs=[
            pl.BlockSpec((1, 1, blk), lambda i: (i, 0, 0),
                         memory_space=pltpu.SMEM),
            pl.BlockSpec((1, 1, blk), lambda i: (i, 0, 0),
                         memory_space=pltpu.SMEM),
            pl.BlockSpec((N, 8, 128), lambda i: (0, 0, 0)),
            pl.BlockSpec((N, 8, 128), lambda i: (0, 0, 0)),
            pl.BlockSpec((blk, 8, 128), lambda i: (i, 0, 0)),
            pl.BlockSpec((8, 128), lambda i: (0, 0)),
        ],
        out_specs=pl.BlockSpec((blk, 8), lambda i: (i, 0)),
        out_shape=jax.ShapeDtypeStruct((E, 8), F32),
        scratch_shapes=[pltpu.VMEM((blk, 8, 128), F32)],
    )(src3, dst3, xlT, xrT, el8, att8)


def _agg_body(src_ref, starts_ref, xl_ref, ex_ref, o_ref, *, n_nodes):
    def node_body(n, carry):
        s0 = starts_ref[0, 0, n]
        s1 = starts_ref[0, 0, n + 1]

        def edge_body(e, uc):
            u, den = uc
            idx = src_ref[0, 0, e]
            exr = ex_ref[e].reshape(1, 8)
            exv = jnp.broadcast_to(exr.T, (8, 128))
            return (u + exv * xl_ref[idx], den + exv)

        u, den = jax.lax.fori_loop(
            s0, s1, edge_body,
            (jnp.zeros((8, 128), F32), jnp.zeros((8, 128), F32)))
        o_ref[n] = u / (den + 1e-16)
        return carry

    jax.lax.fori_loop(0, n_nodes, node_body, 0)


def _agg(src3, starts3, xlT, ex8T):
    N = xlT.shape[0]
    E = ex8T.shape[0]
    return pl.pallas_call(
        functools.partial(_agg_body, n_nodes=N),
        in_specs=[
            pl.BlockSpec(memory_space=pltpu.SMEM),
            pl.BlockSpec(memory_space=pltpu.SMEM),
            pl.BlockSpec((N, 8, 128), lambda: (0, 0, 0)),
            pl.BlockSpec((E, 8), lambda: (0, 0)),
        ],
        out_specs=pl.BlockSpec((N, 8, 128), lambda: (0, 0, 0)),
        out_shape=jax.ShapeDtypeStruct((N, 8, 128), F32),
    )(src3, starts3, xlT, ex8T)


def _trans_body(u_ref, bo_ref, wt_ref, bt_ref, o_ref):
    u = u_ref[...] + bo_ref[...]
    o_ref[...] = jnp.dot(u, wt_ref[...], preferred_element_type=F32) + bt_ref[...]


def _trans(u, bo, wt, bt, blk=512):
    N, HC = u.shape
    DM = wt.shape[1]
    return pl.pallas_call(
        _trans_body,
        grid=(N // blk,),
        in_specs=[
            pl.BlockSpec((blk, HC), lambda i: (i, 0)),
            pl.BlockSpec((1, HC), lambda i: (0, 0)),
            pl.BlockSpec((HC, DM), lambda i: (0, 0)),
            pl.BlockSpec((1, DM), lambda i: (0, 0)),
        ],
        out_specs=pl.BlockSpec((blk, DM), lambda i: (i, 0)),
        out_shape=jax.ShapeDtypeStruct((N, DM), F32),
    )(u, bo, wt, bt)


def _ln(x, g, b):
    m = x.mean(-1, keepdims=True)
    v = ((x - m) ** 2).mean(-1, keepdims=True)
    return (x - m) / jnp.sqrt(v + 1e-5) * g + b


def _gelu(x):
    return x * 0.5 * (1.0 + jax.lax.erf(x / jnp.sqrt(2.0).astype(F32)))


def _final_body(h_ref, batch_ref, lng_ref, lnb_ref, wg_ref, bg_ref,
                wc1_ref, bc1_ref, lncg_ref, lncb_ref,
                wr1_ref, br1_ref, lnrg_ref, lnrb_ref,
                wr2_ref, br2_ref, wc2_ref, bc2_ref, o_ref, *, n_graphs):
    h = _ln(h_ref[...], lng_ref[...], lnb_ref[...])
    gate = jnp.dot(h, wg_ref[...], preferred_element_type=F32) + bg_ref[...]
    N = h.shape[0]
    gi = jax.lax.broadcasted_iota(jnp.int32, (N, n_graphs), 1)
    msk = batch_ref[...] == gi
    gateb = jnp.broadcast_to(gate, (N, n_graphs))
    gm = jnp.max(jnp.where(msk, gateb, -1e30), axis=0, keepdims=True)
    gm = jnp.where(gm < -1e29, 0.0, gm)
    gw = jnp.where(msk, jnp.exp(gateb - gm), 0.0)
    den = jnp.sum(gw, axis=0, keepdims=True)
    ga = gw / (den + 1e-16)
    pooled = jax.lax.dot_general(ga, h, (((0,), (0,)), ((), ())),
                                 preferred_element_type=F32)
    c = jnp.dot(pooled, wc1_ref[...], preferred_element_type=F32) + bc1_ref[...]
    c = _ln(c, lncg_ref[...], lncb_ref[...])
    c = _gelu(c)
    r = jnp.dot(c, wr1_ref[...], preferred_element_type=F32) + br1_ref[...]
    r = _ln(r, lnrg_ref[...], lnrb_ref[...])
    r = _gelu(r)
    r = jnp.dot(r, wr2_ref[...], preferred_element_type=F32) + br2_ref[...]
    c = c + r
    o_ref[...] = jnp.dot(c, wc2_ref[...], preferred_element_type=F32) + bc2_ref[...]


def _final(h, batch2, p, n_graphs):
    row = lambda a: a.reshape(1, -1)
    return pl.pallas_call(
        functools.partial(_final_body, n_graphs=n_graphs),
        out_shape=jax.ShapeDtypeStruct((n_graphs, 1), F32),
    )(h, batch2,
      row(p['ln_pre_g']), row(p['ln_pre_b']), p['Wg'], row(p['bg']),
      p['Wc1'], row(p['bc1']), row(p['ln_c_g']), row(p['ln_c_b']),
      p['Wr1'], row(p['br1']), row(p['ln_r_g']), row(p['ln_r_b']),
      p['Wr2'], row(p['br2']), p['Wc2'], row(p['bc2']))


def kernel(x, edge_index, edge_attr, batch, params):
    G, S, D_IN = x.shape
    N = G * S
    E = edge_index.shape[1]

    src = edge_index[0]
    dst = edge_index[1]
    TIMING_BISECT = 1
    if TIMING_BISECT >= 1:
        perm = jnp.arange(E, dtype=jnp.int32)
    else:
        perm = jnp.argsort(dst)
    dst_s = dst[perm]
    src_s = src[perm]
    ea_s = edge_attr[perm]
    if TIMING_BISECT >= 2:
        starts = (jnp.arange(N + 1, dtype=jnp.int32) * (E // N))
    else:
        starts = jnp.searchsorted(dst_s, jnp.arange(N + 1, dtype=jnp.int32))
    starts = starts.astype(jnp.int32)

    ablk = 1024
    src3 = src_s.reshape(E // ablk, 1, ablk)
    dst3 = dst_s.reshape(E // ablk, 1, ablk)
    src_flat3 = src_s.reshape(1, 1, E)
    starts3 = starts.reshape(1, 1, N + 1)

    row = lambda a: a.reshape(1, -1)
    h = _embed(x, params['W_in'], row(params['b_in']),
               row(params['ln_in_g']), row(params['ln_in_b']),
               params['pe'][:S])

    for lp in params['layers']:
        xl, xr = _proj(h, lp['W_l'], row(lp['b_l']), lp['W_r'], row(lp['b_r']))
        el = _eproj(ea_s, lp['W_e'])
        xlT = xl.reshape(N, 8, 128)
        xrT = xr.reshape(N, 8, 128)
        el8 = el.reshape(E, 8, 128)
        att8 = lp['att'].reshape(8, 128)
        ex8T = _alpha(src3, dst3, xlT, xrT, el8, att8, blk=ablk)
        uT = _agg(src_flat3, starts3, xlT, ex8T)
        h = _trans(uT.reshape(N, 8 * 128), row(lp['b_out']), lp['Wt'],
                   row(lp['bt']))

    batch2 = batch.reshape(N, 1)
    return _final(h, batch2, params, G)


# speedup vs baseline: 1.0265x; 1.0265x over previous
import functools

import jax
import jax.numpy as jnp
from jax.experimental import pallas as pl
from jax.experimental.pallas import tpu as pltpu

F32 = jnp.float32


def _embed_body(x_ref, w_ref, b_ref, g_ref, bb_ref, pe_ref, o_ref):
    h = jnp.dot(x_ref[0], w_ref[...], preferred_element_type=F32) + b_ref[...]
    m = h.mean(-1, keepdims=True)
    v = ((h - m) ** 2).mean(-1, keepdims=True)
    h = (h - m) / jnp.sqrt(v + 1e-5) * g_ref[...] + bb_ref[...]
    o_ref[...] = h + pe_ref[...]


def _embed(x, w, b, g, bb, pe):
    G, S, D_IN = x.shape
    DM = w.shape[1]
    return pl.pallas_call(
        _embed_body,
        grid=(G,),
        in_specs=[
            pl.BlockSpec((1, S, D_IN), lambda i: (i, 0, 0)),
            pl.BlockSpec((D_IN, DM), lambda i: (0, 0)),
            pl.BlockSpec((1, DM), lambda i: (0, 0)),
            pl.BlockSpec((1, DM), lambda i: (0, 0)),
            pl.BlockSpec((1, DM), lambda i: (0, 0)),
            pl.BlockSpec((S, DM), lambda i: (0, 0)),
        ],
        out_specs=pl.BlockSpec((S, DM), lambda i: (i, 0)),
        out_shape=jax.ShapeDtypeStruct((G * S, DM), F32),
    )(x, w, b, g, bb, pe)


def _proj_body(h_ref, wl_ref, bl_ref, wr_ref, br_ref, xl_ref, xr_ref):
    hb = h_ref[...]
    xl_ref[...] = jnp.dot(hb, wl_ref[...], preferred_element_type=F32) + bl_ref[...]
    xr_ref[...] = jnp.dot(hb, wr_ref[...], preferred_element_type=F32) + br_ref[...]


def _proj(h, wl, bl, wr, br, blk=512):
    N, DM = h.shape
    HC = wl.shape[1]
    return pl.pallas_call(
        _proj_body,
        grid=(N // blk,),
        in_specs=[
            pl.BlockSpec((blk, DM), lambda i: (i, 0)),
            pl.BlockSpec((DM, HC), lambda i: (0, 0)),
            pl.BlockSpec((1, HC), lambda i: (0, 0)),
            pl.BlockSpec((DM, HC), lambda i: (0, 0)),
            pl.BlockSpec((1, HC), lambda i: (0, 0)),
        ],
        out_specs=[
            pl.BlockSpec((blk, HC), lambda i: (i, 0)),
            pl.BlockSpec((blk, HC), lambda i: (i, 0)),
        ],
        out_shape=[
            jax.ShapeDtypeStruct((N, HC), F32),
            jax.ShapeDtypeStruct((N, HC), F32),
        ],
    )(h, wl, bl, wr, br)


def _eproj_body(ea_ref, we_ref, o_ref):
    o_ref[...] = jnp.dot(ea_ref[...], we_ref[...], preferred_element_type=F32)


def _eproj(ea, we, blk=2048):
    E, ED = ea.shape
    HC = we.shape[1]
    return pl.pallas_call(
        _eproj_body,
        grid=(E // blk,),
        in_specs=[
            pl.BlockSpec((blk, ED), lambda i: (i, 0)),
            pl.BlockSpec((ED, HC), lambda i: (0, 0)),
        ],
        out_specs=pl.BlockSpec((blk, HC), lambda i: (i, 0)),
        out_shape=jax.ShapeDtypeStruct((E, HC), F32),
    )(ea, we)


def _alpha_body(src_ref, dst_ref, xl_ref, xr_ref, el_ref, att_ref, o_ref,
                matt_ref, *, blk):
    att = att_ref[...]

    def body(j, carry):
        s = src_ref[0, 0, j]
        d = dst_ref[0, 0, j]
        z = xl_ref[s] + xr_ref[d] + el_ref[j]
        z = jnp.maximum(z, 0.2 * z) * att
        matt_ref[j] = z
        return carry

    jax.lax.fori_loop(0, blk, body, 0, unroll=4)
    msum = jnp.sum(matt_ref[...], axis=2)
    p = msum + pltpu.roll(msum, 7, 1)
    lane = jax.lax.broadcasted_iota(jnp.int32, msum.shape, 1)
    a8 = jnp.where(lane % 2 == 0, p, pltpu.roll(p, 1, 1))
    o_ref[...] = jnp.exp(a8)


def _alpha(src3, dst3, xlT, xrT, el8, att8, blk=1024):
    N = xlT.shape[0]
    E = el8.shape[0]
    return pl.pallas_call(
        functools.partial(_alpha_body, blk=blk),
        grid=(E // blk,),
        in_specs=[
            pl.BlockSpec((1, 1, blk), lambda i: (i, 0, 0),
                         memory_space=pltpu.SMEM),
            pl.BlockSpec((1, 1, blk), lambda i: (i, 0, 0),
                         memory_space=pltpu.SMEM),
            pl.BlockSpec((N, 8, 128), lambda i: (0, 0, 0)),
            pl.BlockSpec((N, 8, 128), lambda i: (0, 0, 0)),
            pl.BlockSpec((blk, 8, 128), lambda i: (i, 0, 0)),
            pl.BlockSpec((8, 128), lambda i: (0, 0)),
        ],
        out_specs=pl.BlockSpec((blk, 8), lambda i: (i, 0)),
        out_shape=jax.ShapeDtypeStruct((E, 8), F32),
        scratch_shapes=[pltpu.VMEM((blk, 8, 128), F32)],
    )(src3, dst3, xlT, xrT, el8, att8)


def _agg_body(src_ref, starts_ref, xl_ref, ex_ref, o_ref, *, n_nodes):
    def node_body(n, carry):
        s0 = starts_ref[0, 0, n]
        s1 = starts_ref[0, 0, n + 1]

        def edge_body(e, uc):
            u, den = uc
            idx = src_ref[0, 0, e]
            exr = ex_ref[e].reshape(1, 8)
            exv = jnp.broadcast_to(exr.T, (8, 128))
            return (u + exv * xl_ref[idx], den + exv)

        u, den = jax.lax.fori_loop(
            s0, s1, edge_body,
            (jnp.zeros((8, 128), F32), jnp.zeros((8, 128), F32)))
        o_ref[n] = u / (den + 1e-16)
        return carry

    jax.lax.fori_loop(0, n_nodes, node_body, 0)


def _agg(src3, starts3, xlT, ex8T):
    N = xlT.shape[0]
    E = ex8T.shape[0]
    return pl.pallas_call(
        functools.partial(_agg_body, n_nodes=N),
        in_specs=[
            pl.BlockSpec(memory_space=pltpu.SMEM),
            pl.BlockSpec(memory_space=pltpu.SMEM),
            pl.BlockSpec((N, 8, 128), lambda: (0, 0, 0)),
            pl.BlockSpec((E, 8), lambda: (0, 0)),
        ],
        out_specs=pl.BlockSpec((N, 8, 128), lambda: (0, 0, 0)),
        out_shape=jax.ShapeDtypeStruct((N, 8, 128), F32),
    )(src3, starts3, xlT, ex8T)


def _trans_body(u_ref, bo_ref, wt_ref, bt_ref, o_ref):
    u = u_ref[...] + bo_ref[...]
    o_ref[...] = jnp.dot(u, wt_ref[...], preferred_element_type=F32) + bt_ref[...]


def _trans(u, bo, wt, bt, blk=512):
    N, HC = u.shape
    DM = wt.shape[1]
    return pl.pallas_call(
        _trans_body,
        grid=(N // blk,),
        in_specs=[
            pl.BlockSpec((blk, HC), lambda i: (i, 0)),
            pl.BlockSpec((1, HC), lambda i: (0, 0)),
            pl.BlockSpec((HC, DM), lambda i: (0, 0)),
            pl.BlockSpec((1, DM), lambda i: (0, 0)),
        ],
        out_specs=pl.BlockSpec((blk, DM), lambda i: (i, 0)),
        out_shape=jax.ShapeDtypeStruct((N, DM), F32),
    )(u, bo, wt, bt)


def _ln(x, g, b):
    m = x.mean(-1, keepdims=True)
    v = ((x - m) ** 2).mean(-1, keepdims=True)
    return (x - m) / jnp.sqrt(v + 1e-5) * g + b


def _gelu(x):
    return x * 0.5 * (1.0 + jax.lax.erf(x / jnp.sqrt(2.0).astype(F32)))


def _final_body(h_ref, batch_ref, lng_ref, lnb_ref, wg_ref, bg_ref,
                wc1_ref, bc1_ref, lncg_ref, lncb_ref,
                wr1_ref, br1_ref, lnrg_ref, lnrb_ref,
                wr2_ref, br2_ref, wc2_ref, bc2_ref, o_ref, *, n_graphs):
    h = _ln(h_ref[...], lng_ref[...], lnb_ref[...])
    gate = jnp.dot(h, wg_ref[...], preferred_element_type=F32) + bg_ref[...]
    N = h.shape[0]
    gi = jax.lax.broadcasted_iota(jnp.int32, (N, n_graphs), 1)
    msk = batch_ref[...] == gi
    gateb = jnp.broadcast_to(gate, (N, n_graphs))
    gm = jnp.max(jnp.where(msk, gateb, -1e30), axis=0, keepdims=True)
    gm = jnp.where(gm < -1e29, 0.0, gm)
    gw = jnp.where(msk, jnp.exp(gateb - gm), 0.0)
    den = jnp.sum(gw, axis=0, keepdims=True)
    ga = gw / (den + 1e-16)
    pooled = jax.lax.dot_general(ga, h, (((0,), (0,)), ((), ())),
                                 preferred_element_type=F32)
    c = jnp.dot(pooled, wc1_ref[...], preferred_element_type=F32) + bc1_ref[...]
    c = _ln(c, lncg_ref[...], lncb_ref[...])
    c = _gelu(c)
    r = jnp.dot(c, wr1_ref[...], preferred_element_type=F32) + br1_ref[...]
    r = _ln(r, lnrg_ref[...], lnrb_ref[...])
    r = _gelu(r)
    r = jnp.dot(r, wr2_ref[...], preferred_element_type=F32) + br2_ref[...]
    c = c + r
    o_ref[...] = jnp.dot(c, wc2_ref[...], preferred_element_type=F32) + bc2_ref[...]


def _final(h, batch2, p, n_graphs):
    row = lambda a: a.reshape(1, -1)
    return pl.pallas_call(
        functools.partial(_final_body, n_graphs=n_graphs),
        out_shape=jax.ShapeDtypeStruct((n_graphs, 1), F32),
    )(h, batch2,
      row(p['ln_pre_g']), row(p['ln_pre_b']), p['Wg'], row(p['bg']),
      p['Wc1'], row(p['bc1']), row(p['ln_c_g']), row(p['ln_c_b']),
      p['Wr1'], row(p['br1']), row(p['ln_r_g']), row(p['ln_r_b']),
      p['Wr2'], row(p['br2']), p['Wc2'], row(p['bc2']))


def kernel(x, edge_index, edge_attr, batch, params):
    G, S, D_IN = x.shape
    N = G * S
    E = edge_index.shape[1]

    src = edge_index[0]
    dst = edge_index[1]
    TIMING_BISECT = 2
    if TIMING_BISECT >= 1:
        perm = jnp.arange(E, dtype=jnp.int32)
    else:
        perm = jnp.argsort(dst)
    dst_s = dst[perm]
    src_s = src[perm]
    ea_s = edge_attr[perm]
    if TIMING_BISECT >= 2:
        starts = (jnp.arange(N + 1, dtype=jnp.int32) * (E // N))
    else:
        starts = jnp.searchsorted(dst_s, jnp.arange(N + 1, dtype=jnp.int32))
    starts = starts.astype(jnp.int32)

    ablk = 1024
    src3 = src_s.reshape(E // ablk, 1, ablk)
    dst3 = dst_s.reshape(E // ablk, 1, ablk)
    src_flat3 = src_s.reshape(1, 1, E)
    starts3 = starts.reshape(1, 1, N + 1)

    row = lambda a: a.reshape(1, -1)
    h = _embed(x, params['W_in'], row(params['b_in']),
               row(params['ln_in_g']), row(params['ln_in_b']),
               params['pe'][:S])

    for lp in params['layers']:
        xl, xr = _proj(h, lp['W_l'], row(lp['b_l']), lp['W_r'], row(lp['b_r']))
        el = _eproj(ea_s, lp['W_e'])
        xlT = xl.reshape(N, 8, 128)
        xrT = xr.reshape(N, 8, 128)
        el8 = el.reshape(E, 8, 128)
        att8 = lp['att'].reshape(8, 128)
        ex8T = _alpha(src3, dst3, xlT, xrT, el8, att8, blk=ablk)
        uT = _agg(src_flat3, starts3, xlT, ex8T)
        h = _trans(uT.reshape(N, 8 * 128), row(lp['b_out']), lp['Wt'],
                   row(lp['bt']))

    batch2 = batch.reshape(N, 1)
    return _final(h, batch2, params, G)


# speedup vs baseline: 1.0280x; 1.0014x over previous
import functools

import jax
import jax.numpy as jnp
from jax.experimental import pallas as pl
from jax.experimental.pallas import tpu as pltpu

F32 = jnp.float32


def _embed_body(x_ref, w_ref, b_ref, g_ref, bb_ref, pe_ref, o_ref):
    h = jnp.dot(x_ref[0], w_ref[...], preferred_element_type=F32) + b_ref[...]
    m = h.mean(-1, keepdims=True)
    v = ((h - m) ** 2).mean(-1, keepdims=True)
    h = (h - m) / jnp.sqrt(v + 1e-5) * g_ref[...] + bb_ref[...]
    o_ref[...] = h + pe_ref[...]


def _embed(x, w, b, g, bb, pe):
    G, S, D_IN = x.shape
    DM = w.shape[1]
    return pl.pallas_call(
        _embed_body,
        grid=(G,),
        in_specs=[
            pl.BlockSpec((1, S, D_IN), lambda i: (i, 0, 0)),
            pl.BlockSpec((D_IN, DM), lambda i: (0, 0)),
            pl.BlockSpec((1, DM), lambda i: (0, 0)),
            pl.BlockSpec((1, DM), lambda i: (0, 0)),
            pl.BlockSpec((1, DM), lambda i: (0, 0)),
            pl.BlockSpec((S, DM), lambda i: (0, 0)),
        ],
        out_specs=pl.BlockSpec((S, DM), lambda i: (i, 0)),
        out_shape=jax.ShapeDtypeStruct((G * S, DM), F32),
    )(x, w, b, g, bb, pe)


def _proj_body(h_ref, wl_ref, bl_ref, wr_ref, br_ref, xl_ref, xr_ref):
    hb = h_ref[...]
    xl_ref[...] = jnp.dot(hb, wl_ref[...], preferred_element_type=F32) + bl_ref[...]
    xr_ref[...] = jnp.dot(hb, wr_ref[...], preferred_element_type=F32) + br_ref[...]


def _proj(h, wl, bl, wr, br, blk=512):
    N, DM = h.shape
    HC = wl.shape[1]
    return pl.pallas_call(
        _proj_body,
        grid=(N // blk,),
        in_specs=[
            pl.BlockSpec((blk, DM), lambda i: (i, 0)),
            pl.BlockSpec((DM, HC), lambda i: (0, 0)),
            pl.BlockSpec((1, HC), lambda i: (0, 0)),
            pl.BlockSpec((DM, HC), lambda i: (0, 0)),
            pl.BlockSpec((1, HC), lambda i: (0, 0)),
        ],
        out_specs=[
            pl.BlockSpec((blk, HC), lambda i: (i, 0)),
            pl.BlockSpec((blk, HC), lambda i: (i, 0)),
        ],
        out_shape=[
            jax.ShapeDtypeStruct((N, HC), F32),
            jax.ShapeDtypeStruct((N, HC), F32),
        ],
    )(h, wl, bl, wr, br)


def _eproj_body(ea_ref, we_ref, o_ref):
    o_ref[...] = jnp.dot(ea_ref[...], we_ref[...], preferred_element_type=F32)


def _eproj(ea, we, blk=2048):
    E, ED = ea.shape
    HC = we.shape[1]
    return pl.pallas_call(
        _eproj_body,
        grid=(E // blk,),
        in_specs=[
            pl.BlockSpec((blk, ED), lambda i: (i, 0)),
            pl.BlockSpec((ED, HC), lambda i: (0, 0)),
        ],
        out_specs=pl.BlockSpec((blk, HC), lambda i: (i, 0)),
        out_shape=jax.ShapeDtypeStruct((E, HC), F32),
    )(ea, we)


def _alpha_body(src_ref, dst_ref, xl_ref, xr_ref, el_ref, att_ref, o_ref,
                matt_ref, *, blk):
    att = att_ref[...]

    def body(j, carry):
        s = src_ref[0, 0, j]
        d = dst_ref[0, 0, j]
        z = xl_ref[s] + xr_ref[d] + el_ref[j]
        z = jnp.maximum(z, 0.2 * z) * att
        matt_ref[j] = z
        return carry

    jax.lax.fori_loop(0, blk, body, 0, unroll=4)
    msum = jnp.sum(matt_ref[...], axis=2)
    p = msum + pltpu.roll(msum, 7, 1)
    lane = jax.lax.broadcasted_iota(jnp.int32, msum.shape, 1)
    a8 = jnp.where(lane % 2 == 0, p, pltpu.roll(p, 1, 1))
    o_ref[...] = jnp.exp(a8)


def _alpha(src3, dst3, xlT, xrT, el8, att8, blk=1024):
    N = xlT.shape[0]
    E = el8.shape[0]
    return pl.pallas_call(
        functools.partial(_alpha_body, blk=blk),
        grid=(E // blk,),
        in_specs=[
            pl.BlockSpec((1, 1, blk), lambda i: (i, 0, 0),
                         memory_space=pltpu.SMEM),
            pl.BlockSpec((1, 1, blk), lambda i: (i, 0, 0),
                         memory_space=pltpu.SMEM),
            pl.BlockSpec((N, 8, 128), lambda i: (0, 0, 0)),
            pl.BlockSpec((N, 8, 128), lambda i: (0, 0, 0)),
            pl.BlockSpec((blk, 8, 128), lambda i: (i, 0, 0)),
            pl.BlockSpec((8, 128), lambda i: (0, 0)),
        ],
        out_specs=pl.BlockSpec((blk, 8), lambda i: (i, 0)),
        out_shape=jax.ShapeDtypeStruct((E, 8), F32),
        scratch_shapes=[pltpu.VMEM((blk, 8, 128), F32)],
    )(src3, dst3, xlT, xrT, el8, att8)


def _agg_body(src_ref, starts_ref, xl_ref, ex_ref, o_ref, *, n_nodes):
    def node_body(n, carry):
        s0 = starts_ref[0, 0, n]
        s1 = starts_ref[0, 0, n + 1]

        def edge_body(e, uc):
            u, den = uc
            idx = src_ref[0, 0, e]
            exr = ex_ref[e].reshape(1, 8)
            exv = jnp.broadcast_to(exr.T, (8, 128))
            return (u + exv * xl_ref[idx], den + exv)

        u, den = jax.lax.fori_loop(
            s0, s1, edge_body,
            (jnp.zeros((8, 128), F32), jnp.zeros((8, 128), F32)))
        o_ref[n] = u / (den + 1e-16)
        return carry

    jax.lax.fori_loop(0, n_nodes, node_body, 0)


def _agg(src3, starts3, xlT, ex8T):
    N = xlT.shape[0]
    E = ex8T.shape[0]
    return pl.pallas_call(
        functools.partial(_agg_body, n_nodes=N),
        in_specs=[
            pl.BlockSpec(memory_space=pltpu.SMEM),
            pl.BlockSpec(memory_space=pltpu.SMEM),
            pl.BlockSpec((N, 8, 128), lambda: (0, 0, 0)),
            pl.BlockSpec((E, 8), lambda: (0, 0)),
        ],
        out_specs=pl.BlockSpec((N, 8, 128), lambda: (0, 0, 0)),
        out_shape=jax.ShapeDtypeStruct((N, 8, 128), F32),
    )(src3, starts3, xlT, ex8T)


def _trans_body(u_ref, bo_ref, wt_ref, bt_ref, o_ref):
    u = u_ref[...] + bo_ref[...]
    o_ref[...] = jnp.dot(u, wt_ref[...], preferred_element_type=F32) + bt_ref[...]


def _trans(u, bo, wt, bt, blk=512):
    N, HC = u.shape
    DM = wt.shape[1]
    return pl.pallas_call(
        _trans_body,
        grid=(N // blk,),
        in_specs=[
            pl.BlockSpec((blk, HC), lambda i: (i, 0)),
            pl.BlockSpec((1, HC), lambda i: (0, 0)),
            pl.BlockSpec((HC, DM), lambda i: (0, 0)),
            pl.BlockSpec((1, DM), lambda i: (0, 0)),
        ],
        out_specs=pl.BlockSpec((blk, DM), lambda i: (i, 0)),
        out_shape=jax.ShapeDtypeStruct((N, DM), F32),
    )(u, bo, wt, bt)


def _ln(x, g, b):
    m = x.mean(-1, keepdims=True)
    v = ((x - m) ** 2).mean(-1, keepdims=True)
    return (x - m) / jnp.sqrt(v + 1e-5) * g + b


def _gelu(x):
    return x * 0.5 * (1.0 + jax.lax.erf(x / jnp.sqrt(2.0).astype(F32)))


def _final_body(h_ref, batch_ref, lng_ref, lnb_ref, wg_ref, bg_ref,
                wc1_ref, bc1_ref, lncg_ref, lncb_ref,
                wr1_ref, br1_ref, lnrg_ref, lnrb_ref,
                wr2_ref, br2_ref, wc2_ref, bc2_ref, o_ref, *, n_graphs):
    h = _ln(h_ref[...], lng_ref[...], lnb_ref[...])
    gate = jnp.dot(h, wg_ref[...], preferred_element_type=F32) + bg_ref[...]
    N = h.shape[0]
    gi = jax.lax.broadcasted_iota(jnp.int32, (N, n_graphs), 1)
    msk = batch_ref[...] == gi
    gateb = jnp.broadcast_to(gate, (N, n_graphs))
    gm = jnp.max(jnp.where(msk, gateb, -1e30), axis=0, keepdims=True)
    gm = jnp.where(gm < -1e29, 0.0, gm)
    gw = jnp.where(msk, jnp.exp(gateb - gm), 0.0)
    den = jnp.sum(gw, axis=0, keepdims=True)
    ga = gw / (den + 1e-16)
    pooled = jax.lax.dot_general(ga, h, (((0,), (0,)), ((), ())),
                                 preferred_element_type=F32)
    c = jnp.dot(pooled, wc1_ref[...], preferred_element_type=F32) + bc1_ref[...]
    c = _ln(c, lncg_ref[...], lncb_ref[...])
    c = _gelu(c)
    r = jnp.dot(c, wr1_ref[...], preferred_element_type=F32) + br1_ref[...]
    r = _ln(r, lnrg_ref[...], lnrb_ref[...])
    r = _gelu(r)
    r = jnp.dot(r, wr2_ref[...], preferred_element_type=F32) + br2_ref[...]
    c = c + r
    o_ref[...] = jnp.dot(c, wc2_ref[...], preferred_element_type=F32) + bc2_ref[...]


def _final(h, batch2, p, n_graphs):
    row = lambda a: a.reshape(1, -1)
    return pl.pallas_call(
        functools.partial(_final_body, n_graphs=n_graphs),
        out_shape=jax.ShapeDtypeStruct((n_graphs, 1), F32),
    )(h, batch2,
      row(p['ln_pre_g']), row(p['ln_pre_b']), p['Wg'], row(p['bg']),
      p['Wc1'], row(p['bc1']), row(p['ln_c_g']), row(p['ln_c_b']),
      p['Wr1'], row(p['br1']), row(p['ln_r_g']), row(p['ln_r_b']),
      p['Wr2'], row(p['br2']), p['Wc2'], row(p['bc2']))


def kernel(x, edge_index, edge_attr, batch, params):
    G, S, D_IN = x.shape
    N = G * S
    E = edge_index.shape[1]

    src = edge_index[0]
    dst = edge_index[1]
    TIMING_BISECT = 3
    if TIMING_BISECT >= 3:
        dst_s, src_s, ea_s = dst, src, edge_attr
    else:
        if TIMING_BISECT >= 1:
            perm = jnp.arange(E, dtype=jnp.int32)
        else:
            perm = jnp.argsort(dst)
        dst_s = dst[perm]
        src_s = src[perm]
        ea_s = edge_attr[perm]
    if TIMING_BISECT >= 2:
        starts = (jnp.arange(N + 1, dtype=jnp.int32) * (E // N))
    else:
        starts = jnp.searchsorted(dst_s, jnp.arange(N + 1, dtype=jnp.int32))
    starts = starts.astype(jnp.int32)

    ablk = 1024
    src3 = src_s.reshape(E // ablk, 1, ablk)
    dst3 = dst_s.reshape(E // ablk, 1, ablk)
    src_flat3 = src_s.reshape(1, 1, E)
    starts3 = starts.reshape(1, 1, N + 1)

    row = lambda a: a.reshape(1, -1)
    h = _embed(x, params['W_in'], row(params['b_in']),
               row(params['ln_in_g']), row(params['ln_in_b']),
               params['pe'][:S])

    for lp in params['layers']:
        xl, xr = _proj(h, lp['W_l'], row(lp['b_l']), lp['W_r'], row(lp['b_r']))
        el = _eproj(ea_s, lp['W_e'])
        xlT = xl.reshape(N, 8, 128)
        xrT = xr.reshape(N, 8, 128)
        el8 = el.reshape(E, 8, 128)
        att8 = lp['att'].reshape(8, 128)
        ex8T = _alpha(src3, dst3, xlT, xrT, el8, att8, blk=ablk)
        uT = _agg(src_flat3, starts3, xlT, ex8T)
        h = _trans(uT.reshape(N, 8 * 128), row(lp['b_out']), lp['Wt'],
                   row(lp['bt']))

    batch2 = batch.reshape(N, 1)
    return _final(h, batch2, params, G)


# speedup vs baseline: 7.9305x; 7.7146x over previous
import functools

import jax
import jax.numpy as jnp
from jax.experimental import pallas as pl
from jax.experimental.pallas import tpu as pltpu

F32 = jnp.float32


def _embed_body(x_ref, w_ref, b_ref, g_ref, bb_ref, pe_ref, o_ref):
    h = jnp.dot(x_ref[0], w_ref[...], preferred_element_type=F32) + b_ref[...]
    m = h.mean(-1, keepdims=True)
    v = ((h - m) ** 2).mean(-1, keepdims=True)
    h = (h - m) / jnp.sqrt(v + 1e-5) * g_ref[...] + bb_ref[...]
    o_ref[...] = h + pe_ref[...]


def _embed(x, w, b, g, bb, pe):
    G, S, D_IN = x.shape
    DM = w.shape[1]
    return pl.pallas_call(
        _embed_body,
        grid=(G,),
        in_specs=[
            pl.BlockSpec((1, S, D_IN), lambda i: (i, 0, 0)),
            pl.BlockSpec((D_IN, DM), lambda i: (0, 0)),
            pl.BlockSpec((1, DM), lambda i: (0, 0)),
            pl.BlockSpec((1, DM), lambda i: (0, 0)),
            pl.BlockSpec((1, DM), lambda i: (0, 0)),
            pl.BlockSpec((S, DM), lambda i: (0, 0)),
        ],
        out_specs=pl.BlockSpec((S, DM), lambda i: (i, 0)),
        out_shape=jax.ShapeDtypeStruct((G * S, DM), F32),
    )(x, w, b, g, bb, pe)


def _proj_body(h_ref, wl_ref, bl_ref, wr_ref, br_ref, xl_ref, xr_ref):
    hb = h_ref[...]
    xl_ref[...] = jnp.dot(hb, wl_ref[...], preferred_element_type=F32) + bl_ref[...]
    xr_ref[...] = jnp.dot(hb, wr_ref[...], preferred_element_type=F32) + br_ref[...]


def _proj(h, wl, bl, wr, br, blk=512):
    N, DM = h.shape
    HC = wl.shape[1]
    return pl.pallas_call(
        _proj_body,
        grid=(N // blk,),
        in_specs=[
            pl.BlockSpec((blk, DM), lambda i: (i, 0)),
            pl.BlockSpec((DM, HC), lambda i: (0, 0)),
            pl.BlockSpec((1, HC), lambda i: (0, 0)),
            pl.BlockSpec((DM, HC), lambda i: (0, 0)),
            pl.BlockSpec((1, HC), lambda i: (0, 0)),
        ],
        out_specs=[
            pl.BlockSpec((blk, HC), lambda i: (i, 0)),
            pl.BlockSpec((blk, HC), lambda i: (i, 0)),
        ],
        out_shape=[
            jax.ShapeDtypeStruct((N, HC), F32),
            jax.ShapeDtypeStruct((N, HC), F32),
        ],
    )(h, wl, bl, wr, br)


def _eproj_body(ea_ref, we_ref, o_ref):
    o_ref[...] = jnp.dot(ea_ref[...], we_ref[...], preferred_element_type=F32)


def _eproj(ea, we, blk=2048):
    E, ED = ea.shape
    HC = we.shape[1]
    return pl.pallas_call(
        _eproj_body,
        grid=(E // blk,),
        in_specs=[
            pl.BlockSpec((blk, ED), lambda i: (i, 0)),
            pl.BlockSpec((ED, HC), lambda i: (0, 0)),
        ],
        out_specs=pl.BlockSpec((blk, HC), lambda i: (i, 0)),
        out_shape=jax.ShapeDtypeStruct((E, HC), F32),
    )(ea, we)


def _alpha_body(src_ref, dst_ref, xl_ref, xr_ref, el_ref, att_ref, o_ref,
                matt_ref, *, blk):
    att = att_ref[...]

    def body(j, carry):
        s = src_ref[0, 0, j]
        d = dst_ref[0, 0, j]
        z = xl_ref[s] + xr_ref[d] + el_ref[j]
        z = jnp.maximum(z, 0.2 * z) * att
        matt_ref[j] = z
        return carry

    jax.lax.fori_loop(0, blk, body, 0, unroll=4)
    msum = jnp.sum(matt_ref[...], axis=2)
    p = msum + pltpu.roll(msum, 7, 1)
    lane = jax.lax.broadcasted_iota(jnp.int32, msum.shape, 1)
    a8 = jnp.where(lane % 2 == 0, p, pltpu.roll(p, 1, 1))
    o_ref[...] = jnp.exp(a8.T)


def _alpha(src3, dst3, xlT, xrT, el8, att8, blk=1024):
    N = xlT.shape[0]
    E = el8.shape[0]
    return pl.pallas_call(
        functools.partial(_alpha_body, blk=blk),
        grid=(E // blk,),
        in_specs=[
            pl.BlockSpec((1, 1, blk), lambda i: (i, 0, 0),
                         memory_space=pltpu.SMEM),
            pl.BlockSpec((1, 1, blk), lambda i: (i, 0, 0),
                         memory_space=pltpu.SMEM),
            pl.BlockSpec((N, 8, 128), lambda i: (0, 0, 0)),
            pl.BlockSpec((N, 8, 128), lambda i: (0, 0, 0)),
            pl.BlockSpec((blk, 8, 128), lambda i: (i, 0, 0)),
            pl.BlockSpec((8, 128), lambda i: (0, 0)),
        ],
        out_specs=pl.BlockSpec((8, blk), lambda i: (0, i)),
        out_shape=jax.ShapeDtypeStruct((8, E), F32),
        scratch_shapes=[pltpu.VMEM((blk, 8, 128), F32)],
    )(src3, dst3, xlT, xrT, el8, att8)


def _agg_body(src_ref, dst_ref, xl_ref, ex_ref, o_ref, den_ref, *,
              n_edges, chunk=128):
    zero = jnp.zeros(o_ref.shape, F32)
    o_ref[...] = zero
    den_ref[...] = zero

    def chunk_body(c, carry):
        base = pl.multiple_of(c * chunk, chunk)
        exc = ex_ref[:, pl.ds(base, chunk)]
        for j in range(chunk):
            u, den, d_prev = carry
            e = base + j
            idx = src_ref[0, 0, e]
            d = dst_ref[0, 0, e]
            exv = jnp.broadcast_to(exc[:, j:j + 1], (8, 128))
            m = jnp.where(d == d_prev, 1.0, 0.0).astype(F32)
            u = u * m + exv * xl_ref[idx]
            den = den * m + exv
            o_ref[d] = u
            den_ref[d] = den
            carry = (u, den, d)
        return carry

    jax.lax.fori_loop(
        0, n_edges // chunk, chunk_body,
        (jnp.zeros((8, 128), F32), jnp.zeros((8, 128), F32),
         jnp.int32(-1)))
    o_ref[...] = o_ref[...] / (den_ref[...] + 1e-16)


def _agg(src3, dst3, xlT, ex8T):
    N = xlT.shape[0]
    E = ex8T.shape[1]
    return pl.pallas_call(
        functools.partial(_agg_body, n_edges=E),
        in_specs=[
            pl.BlockSpec(memory_space=pltpu.SMEM),
            pl.BlockSpec(memory_space=pltpu.SMEM),
            pl.BlockSpec((N, 8, 128), lambda: (0, 0, 0)),
            pl.BlockSpec((8, E), lambda: (0, 0)),
        ],
        out_specs=pl.BlockSpec((N, 8, 128), lambda: (0, 0, 0)),
        out_shape=jax.ShapeDtypeStruct((N, 8, 128), F32),
        scratch_shapes=[pltpu.VMEM((N, 8, 128), F32)],
    )(src3, dst3, xlT, ex8T)


def _trans_body(u_ref, bo_ref, wt_ref, bt_ref, o_ref):
    u = u_ref[...] + bo_ref[...]
    o_ref[...] = jnp.dot(u, wt_ref[...], preferred_element_type=F32) + bt_ref[...]


def _trans(u, bo, wt, bt, blk=512):
    N, HC = u.shape
    DM = wt.shape[1]
    return pl.pallas_call(
        _trans_body,
        grid=(N // blk,),
        in_specs=[
            pl.BlockSpec((blk, HC), lambda i: (i, 0)),
            pl.BlockSpec((1, HC), lambda i: (0, 0)),
            pl.BlockSpec((HC, DM), lambda i: (0, 0)),
            pl.BlockSpec((1, DM), lambda i: (0, 0)),
        ],
        out_specs=pl.BlockSpec((blk, DM), lambda i: (i, 0)),
        out_shape=jax.ShapeDtypeStruct((N, DM), F32),
    )(u, bo, wt, bt)


def _ln(x, g, b):
    m = x.mean(-1, keepdims=True)
    v = ((x - m) ** 2).mean(-1, keepdims=True)
    return (x - m) / jnp.sqrt(v + 1e-5) * g + b


def _gelu(x):
    return x * 0.5 * (1.0 + jax.lax.erf(x / jnp.sqrt(2.0).astype(F32)))


def _final_body(h_ref, batch_ref, lng_ref, lnb_ref, wg_ref, bg_ref,
                wc1_ref, bc1_ref, lncg_ref, lncb_ref,
                wr1_ref, br1_ref, lnrg_ref, lnrb_ref,
                wr2_ref, br2_ref, wc2_ref, bc2_ref, o_ref, *, n_graphs):
    h = _ln(h_ref[...], lng_ref[...], lnb_ref[...])
    gate = jnp.dot(h, wg_ref[...], preferred_element_type=F32) + bg_ref[...]
    N = h.shape[0]
    gi = jax.lax.broadcasted_iota(jnp.int32, (N, n_graphs), 1)
    msk = batch_ref[...] == gi
    gateb = jnp.broadcast_to(gate, (N, n_graphs))
    gm = jnp.max(jnp.where(msk, gateb, -1e30), axis=0, keepdims=True)
    gm = jnp.where(gm < -1e29, 0.0, gm)
    gw = jnp.where(msk, jnp.exp(gateb - gm), 0.0)
    den = jnp.sum(gw, axis=0, keepdims=True)
    ga = gw / (den + 1e-16)
    pooled = jax.lax.dot_general(ga, h, (((0,), (0,)), ((), ())),
                                 preferred_element_type=F32)
    c = jnp.dot(pooled, wc1_ref[...], preferred_element_type=F32) + bc1_ref[...]
    c = _ln(c, lncg_ref[...], lncb_ref[...])
    c = _gelu(c)
    r = jnp.dot(c, wr1_ref[...], preferred_element_type=F32) + br1_ref[...]
    r = _ln(r, lnrg_ref[...], lnrb_ref[...])
    r = _gelu(r)
    r = jnp.dot(r, wr2_ref[...], preferred_element_type=F32) + br2_ref[...]
    c = c + r
    o_ref[...] = jnp.dot(c, wc2_ref[...], preferred_element_type=F32) + bc2_ref[...]


def _final(h, batch2, p, n_graphs):
    row = lambda a: a.reshape(1, -1)
    return pl.pallas_call(
        functools.partial(_final_body, n_graphs=n_graphs),
        out_shape=jax.ShapeDtypeStruct((n_graphs, 1), F32),
    )(h, batch2,
      row(p['ln_pre_g']), row(p['ln_pre_b']), p['Wg'], row(p['bg']),
      p['Wc1'], row(p['bc1']), row(p['ln_c_g']), row(p['ln_c_b']),
      p['Wr1'], row(p['br1']), row(p['ln_r_g']), row(p['ln_r_b']),
      p['Wr2'], row(p['br2']), p['Wc2'], row(p['bc2']))


def kernel(x, edge_index, edge_attr, batch, params):
    G, S, D_IN = x.shape
    N = G * S
    E = edge_index.shape[1]

    src = edge_index[0]
    dst = edge_index[1]
    perm = jnp.argsort(dst)
    dst_s = dst[perm]
    src_s = src[perm]
    ea_s = edge_attr[perm]

    ablk = 1024
    src3 = src_s.reshape(E // ablk, 1, ablk)
    dst3 = dst_s.reshape(E // ablk, 1, ablk)
    src_flat3 = src_s.reshape(1, 1, E)
    dst_flat3 = dst_s.reshape(1, 1, E)

    row = lambda a: a.reshape(1, -1)
    h = _embed(x, params['W_in'], row(params['b_in']),
               row(params['ln_in_g']), row(params['ln_in_b']),
               params['pe'][:S])

    for lp in params['layers']:
        xl, xr = _proj(h, lp['W_l'], row(lp['b_l']), lp['W_r'], row(lp['b_r']))
        el = _eproj(ea_s, lp['W_e'])
        xlT = xl.reshape(N, 8, 128)
        xrT = xr.reshape(N, 8, 128)
        el8 = el.reshape(E, 8, 128)
        att8 = lp['att'].reshape(8, 128)
        ex8T = _alpha(src3, dst3, xlT, xrT, el8, att8, blk=ablk)
        uT = _agg(src_flat3, dst_flat3, xlT, ex8T)
        h = _trans(uT.reshape(N, 8 * 128), row(lp['b_out']), lp['Wt'],
                   row(lp['bt']))

    batch2 = batch.reshape(N, 1)
    return _final(h, batch2, params, G)


# speedup vs baseline: 9.7616x; 1.2309x over previous
import functools

import jax
import jax.numpy as jnp
from jax.experimental import pallas as pl
from jax.experimental.pallas import tpu as pltpu

F32 = jnp.float32
BF16 = jnp.bfloat16


def _dot(a, b):
    return jnp.dot(a.astype(BF16), b.astype(BF16),
                   preferred_element_type=F32)


def _embed_body(x_ref, w_ref, b_ref, g_ref, bb_ref, pe_ref, o_ref):
    h = _dot(x_ref[0], w_ref[...]) + b_ref[...]
    m = h.mean(-1, keepdims=True)
    v = ((h - m) ** 2).mean(-1, keepdims=True)
    h = (h - m) / jnp.sqrt(v + 1e-5) * g_ref[...] + bb_ref[...]
    o_ref[...] = h + pe_ref[...]


def _embed(x, w, b, g, bb, pe):
    G, S, D_IN = x.shape
    DM = w.shape[1]
    return pl.pallas_call(
        _embed_body,
        grid=(G,),
        in_specs=[
            pl.BlockSpec((1, S, D_IN), lambda i: (i, 0, 0)),
            pl.BlockSpec((D_IN, DM), lambda i: (0, 0)),
            pl.BlockSpec((1, DM), lambda i: (0, 0)),
            pl.BlockSpec((1, DM), lambda i: (0, 0)),
            pl.BlockSpec((1, DM), lambda i: (0, 0)),
            pl.BlockSpec((S, DM), lambda i: (0, 0)),
        ],
        out_specs=pl.BlockSpec((S, DM), lambda i: (i, 0)),
        out_shape=jax.ShapeDtypeStruct((G * S, DM), F32),
    )(x, w, b, g, bb, pe)


def _proj_body(h_ref, wl_ref, bl_ref, wr_ref, br_ref, xl_ref, xr_ref):
    hb = h_ref[...]
    xl_ref[...] = _dot(hb, wl_ref[...]) + bl_ref[...]
    xr_ref[...] = _dot(hb, wr_ref[...]) + br_ref[...]


def _proj(h, wl, bl, wr, br, blk=512):
    N, DM = h.shape
    HC = wl.shape[1]
    return pl.pallas_call(
        _proj_body,
        grid=(N // blk,),
        in_specs=[
            pl.BlockSpec((blk, DM), lambda i: (i, 0)),
            pl.BlockSpec((DM, HC), lambda i: (0, 0)),
            pl.BlockSpec((1, HC), lambda i: (0, 0)),
            pl.BlockSpec((DM, HC), lambda i: (0, 0)),
            pl.BlockSpec((1, HC), lambda i: (0, 0)),
        ],
        out_specs=[
            pl.BlockSpec((blk, HC), lambda i: (i, 0)),
            pl.BlockSpec((blk, HC), lambda i: (i, 0)),
        ],
        out_shape=[
            jax.ShapeDtypeStruct((N, HC), F32),
            jax.ShapeDtypeStruct((N, HC), F32),
        ],
    )(h, wl, bl, wr, br)


def _eproj_body(ea_ref, we_ref, o_ref):
    o_ref[...] = _dot(ea_ref[...], we_ref[...])


def _eproj(ea, we, blk=2048):
    E, ED = ea.shape
    HC = we.shape[1]
    return pl.pallas_call(
        _eproj_body,
        grid=(E // blk,),
        in_specs=[
            pl.BlockSpec((blk, ED), lambda i: (i, 0)),
            pl.BlockSpec((ED, HC), lambda i: (0, 0)),
        ],
        out_specs=pl.BlockSpec((blk, HC), lambda i: (i, 0)),
        out_shape=jax.ShapeDtypeStruct((E, HC), F32),
    )(ea, we)


def _alpha_body(src_ref, dst_ref, xl_ref, xr_ref, el_ref, att_ref, o_ref,
                matt_ref, *, blk):
    att = att_ref[...]

    def body(j, carry):
        s = src_ref[0, 0, j]
        d = dst_ref[0, 0, j]
        z = xl_ref[s] + xr_ref[d] + el_ref[j]
        z = jnp.maximum(z, 0.2 * z) * att
        matt_ref[j] = z
        return carry

    jax.lax.fori_loop(0, blk, body, 0, unroll=8)
    msum = jnp.sum(matt_ref[...], axis=2)
    t = msum.T
    p = t + pltpu.roll(t, 7, 0)
    sub = jax.lax.broadcasted_iota(jnp.int32, t.shape, 0)
    a8 = jnp.where(sub % 2 == 0, p, pltpu.roll(p, 1, 0))
    o_ref[...] = jnp.exp(a8)


def _alpha(src3, dst3, xlT, xrT, el8, att8, blk=1024):
    N = xlT.shape[0]
    E = el8.shape[0]
    return pl.pallas_call(
        functools.partial(_alpha_body, blk=blk),
        grid=(E // blk,),
        in_specs=[
            pl.BlockSpec((1, 1, blk), lambda i: (i, 0, 0),
                         memory_space=pltpu.SMEM),
            pl.BlockSpec((1, 1, blk), lambda i: (i, 0, 0),
                         memory_space=pltpu.SMEM),
            pl.BlockSpec((N, 8, 128), lambda i: (0, 0, 0)),
            pl.BlockSpec((N, 8, 128), lambda i: (0, 0, 0)),
            pl.BlockSpec((blk, 8, 128), lambda i: (i, 0, 0)),
            pl.BlockSpec((8, 128), lambda i: (0, 0)),
        ],
        out_specs=pl.BlockSpec((8, blk), lambda i: (0, i)),
        out_shape=jax.ShapeDtypeStruct((8, E), F32),
        scratch_shapes=[pltpu.VMEM((blk, 8, 128), F32)],
    )(src3, dst3, xlT, xrT, el8, att8)


def _agg_body(src_ref, dst_ref, xl_ref, ex_ref, o_ref, den_ref, *,
              n_edges, chunk=128):
    zero = jnp.zeros(o_ref.shape, F32)
    o_ref[...] = zero
    den_ref[...] = zero

    def chunk_body(c, carry):
        base = pl.multiple_of(c * chunk, chunk)
        exc = ex_ref[:, pl.ds(base, chunk)]
        for j in range(chunk):
            u, den, d_prev = carry
            e = base + j
            idx = src_ref[0, 0, e]
            d = dst_ref[0, 0, e]
            exv = jnp.broadcast_to(exc[:, j:j + 1], (8, 128))
            m = jnp.where(d == d_prev, 1.0, 0.0).astype(F32)
            u = u * m + exv * xl_ref[idx]
            den = den * m + exv
            o_ref[d] = u
            den_ref[d] = den
            carry = (u, den, d)
        return carry

    jax.lax.fori_loop(
        0, n_edges // chunk, chunk_body,
        (jnp.zeros((8, 128), F32), jnp.zeros((8, 128), F32),
         jnp.int32(-1)))
    o_ref[...] = o_ref[...] / (den_ref[...] + 1e-16)


def _agg(src3, dst3, xlT, ex8T):
    N = xlT.shape[0]
    E = ex8T.shape[1]
    return pl.pallas_call(
        functools.partial(_agg_body, n_edges=E),
        in_specs=[
            pl.BlockSpec(memory_space=pltpu.SMEM),
            pl.BlockSpec(memory_space=pltpu.SMEM),
            pl.BlockSpec((N, 8, 128), lambda: (0, 0, 0)),
            pl.BlockSpec((8, E), lambda: (0, 0)),
        ],
        out_specs=pl.BlockSpec((N, 8, 128), lambda: (0, 0, 0)),
        out_shape=jax.ShapeDtypeStruct((N, 8, 128), F32),
        scratch_shapes=[pltpu.VMEM((N, 8, 128), F32)],
    )(src3, dst3, xlT, ex8T)


def _trans_body(u_ref, bo_ref, wt_ref, bt_ref, o_ref):
    u = u_ref[...] + bo_ref[...]
    o_ref[...] = _dot(u, wt_ref[...]) + bt_ref[...]


def _trans(u, bo, wt, bt, blk=512):
    N, HC = u.shape
    DM = wt.shape[1]
    return pl.pallas_call(
        _trans_body,
        grid=(N // blk,),
        in_specs=[
            pl.BlockSpec((blk, HC), lambda i: (i, 0)),
            pl.BlockSpec((1, HC), lambda i: (0, 0)),
            pl.BlockSpec((HC, DM), lambda i: (0, 0)),
            pl.BlockSpec((1, DM), lambda i: (0, 0)),
        ],
        out_specs=pl.BlockSpec((blk, DM), lambda i: (i, 0)),
        out_shape=jax.ShapeDtypeStruct((N, DM), F32),
    )(u, bo, wt, bt)


def _ln(x, g, b):
    m = x.mean(-1, keepdims=True)
    v = ((x - m) ** 2).mean(-1, keepdims=True)
    return (x - m) / jnp.sqrt(v + 1e-5) * g + b


def _gelu(x):
    return x * 0.5 * (1.0 + jax.lax.erf(x / jnp.sqrt(2.0).astype(F32)))


def _final_body(h_ref, batch_ref, lng_ref, lnb_ref, wg_ref, bg_ref,
                wc1_ref, bc1_ref, lncg_ref, lncb_ref,
                wr1_ref, br1_ref, lnrg_ref, lnrb_ref,
                wr2_ref, br2_ref, wc2_ref, bc2_ref, o_ref, *, n_graphs):
    h = _ln(h_ref[...], lng_ref[...], lnb_ref[...])
    gate = _dot(h, wg_ref[...]) + bg_ref[...]
    N = h.shape[0]
    gi = jax.lax.broadcasted_iota(jnp.int32, (N, n_graphs), 1)
    msk = batch_ref[...] == gi
    gateb = jnp.broadcast_to(gate, (N, n_graphs))
    gm = jnp.max(jnp.where(msk, gateb, -1e30), axis=0, keepdims=True)
    gm = jnp.where(gm < -1e29, 0.0, gm)
    gw = jnp.where(msk, jnp.exp(gateb - gm), 0.0)
    den = jnp.sum(gw, axis=0, keepdims=True)
    ga = gw / (den + 1e-16)
    pooled = jnp.concatenate(
        [jnp.sum(ga[:, g:g + 1] * h, axis=0, keepdims=True)
         for g in range(n_graphs)], axis=0)
    c = _dot(pooled, wc1_ref[...]) + bc1_ref[...]
    c = _ln(c, lncg_ref[...], lncb_ref[...])
    c = _gelu(c)
    r = _dot(c, wr1_ref[...]) + br1_ref[...]
    r = _ln(r, lnrg_ref[...], lnrb_ref[...])
    r = _gelu(r)
    r = _dot(r, wr2_ref[...]) + br2_ref[...]
    c = c + r
    o_ref[...] = _dot(c, wc2_ref[...]) + bc2_ref[...]


def _final(h, batch2, p, n_graphs):
    row = lambda a: a.reshape(1, -1)
    return pl.pallas_call(
        functools.partial(_final_body, n_graphs=n_graphs),
        out_shape=jax.ShapeDtypeStruct((n_graphs, 1), F32),
    )(h, batch2,
      row(p['ln_pre_g']), row(p['ln_pre_b']), p['Wg'], row(p['bg']),
      p['Wc1'], row(p['bc1']), row(p['ln_c_g']), row(p['ln_c_b']),
      p['Wr1'], row(p['br1']), row(p['ln_r_g']), row(p['ln_r_b']),
      p['Wr2'], row(p['br2']), p['Wc2'], row(p['bc2']))


def kernel(x, edge_index, edge_attr, batch, params):
    G, S, D_IN = x.shape
    N = G * S
    E = edge_index.shape[1]

    src = edge_index[0]
    dst = edge_index[1]
    perm = jnp.argsort(dst)
    dst_s = dst[perm]
    src_s = src[perm]
    ea_s = edge_attr[perm]

    ablk = 1024
    src3 = src_s.reshape(E // ablk, 1, ablk)
    dst3 = dst_s.reshape(E // ablk, 1, ablk)
    src_flat3 = src_s.reshape(1, 1, E)
    dst_flat3 = dst_s.reshape(1, 1, E)

    row = lambda a: a.reshape(1, -1)
    h = _embed(x, params['W_in'], row(params['b_in']),
               row(params['ln_in_g']), row(params['ln_in_b']),
               params['pe'][:S])

    for lp in params['layers']:
        xl, xr = _proj(h, lp['W_l'], row(lp['b_l']), lp['W_r'], row(lp['b_r']))
        el = _eproj(ea_s, lp['W_e'])
        xlT = xl.reshape(N, 8, 128)
        xrT = xr.reshape(N, 8, 128)
        el8 = el.reshape(E, 8, 128)
        att8 = lp['att'].reshape(8, 128)
        ex8T = _alpha(src3, dst3, xlT, xrT, el8, att8, blk=ablk)
        uT = _agg(src_flat3, dst_flat3, xlT, ex8T)
        h = _trans(uT.reshape(N, 8 * 128), row(lp['b_out']), lp['Wt'],
                   row(lp['bt']))

    batch2 = batch.reshape(N, 1)
    return _final(h, batch2, params, G)


# speedup vs baseline: 9.7818x; 1.0021x over previous
import functools

import jax
import jax.numpy as jnp
from jax.experimental import pallas as pl
from jax.experimental.pallas import tpu as pltpu

F32 = jnp.float32
BF16 = jnp.bfloat16


def _dot(a, b):
    return jnp.dot(a.astype(BF16), b.astype(BF16),
                   preferred_element_type=F32)


def _embed_body(x_ref, w_ref, b_ref, g_ref, bb_ref, pe_ref, o_ref):
    h = _dot(x_ref[0], w_ref[...]) + b_ref[...]
    m = h.mean(-1, keepdims=True)
    v = ((h - m) ** 2).mean(-1, keepdims=True)
    h = (h - m) / jnp.sqrt(v + 1e-5) * g_ref[...] + bb_ref[...]
    o_ref[...] = h + pe_ref[...]


def _embed(x, w, b, g, bb, pe):
    G, S, D_IN = x.shape
    DM = w.shape[1]
    return pl.pallas_call(
        _embed_body,
        grid=(G,),
        in_specs=[
            pl.BlockSpec((1, S, D_IN), lambda i: (i, 0, 0)),
            pl.BlockSpec((D_IN, DM), lambda i: (0, 0)),
            pl.BlockSpec((1, DM), lambda i: (0, 0)),
            pl.BlockSpec((1, DM), lambda i: (0, 0)),
            pl.BlockSpec((1, DM), lambda i: (0, 0)),
            pl.BlockSpec((S, DM), lambda i: (0, 0)),
        ],
        out_specs=pl.BlockSpec((S, DM), lambda i: (i, 0)),
        out_shape=jax.ShapeDtypeStruct((G * S, DM), F32),
    )(x, w, b, g, bb, pe)


def _proj_body(h_ref, wl_ref, bl_ref, wr_ref, br_ref, xl_ref, xr_ref):
    hb = h_ref[...]
    xl_ref[...] = _dot(hb, wl_ref[...]) + bl_ref[...]
    xr_ref[...] = _dot(hb, wr_ref[...]) + br_ref[...]


def _proj(h, wl, bl, wr, br, blk=512):
    N, DM = h.shape
    HC = wl.shape[1]
    return pl.pallas_call(
        _proj_body,
        grid=(N // blk,),
        in_specs=[
            pl.BlockSpec((blk, DM), lambda i: (i, 0)),
            pl.BlockSpec((DM, HC), lambda i: (0, 0)),
            pl.BlockSpec((1, HC), lambda i: (0, 0)),
            pl.BlockSpec((DM, HC), lambda i: (0, 0)),
            pl.BlockSpec((1, HC), lambda i: (0, 0)),
        ],
        out_specs=[
            pl.BlockSpec((blk, HC), lambda i: (i, 0)),
            pl.BlockSpec((blk, HC), lambda i: (i, 0)),
        ],
        out_shape=[
            jax.ShapeDtypeStruct((N, HC), F32),
            jax.ShapeDtypeStruct((N, HC), F32),
        ],
    )(h, wl, bl, wr, br)


def _eproj_body(ea_ref, we_ref, o_ref):
    o_ref[...] = _dot(ea_ref[...], we_ref[...])


def _eproj(ea, we, blk=2048):
    E, ED = ea.shape
    HC = we.shape[1]
    return pl.pallas_call(
        _eproj_body,
        grid=(E // blk,),
        in_specs=[
            pl.BlockSpec((blk, ED), lambda i: (i, 0)),
            pl.BlockSpec((ED, HC), lambda i: (0, 0)),
        ],
        out_specs=pl.BlockSpec((blk, HC), lambda i: (i, 0)),
        out_shape=jax.ShapeDtypeStruct((E, HC), F32),
    )(ea, we)


def _alpha_body(src_ref, dst_ref, xl_ref, xr_ref, el_ref, att_ref, o_ref,
                matt_ref, *, blk):
    att = att_ref[...]

    def body(j, carry):
        s = src_ref[0, 0, j]
        d = dst_ref[0, 0, j]
        z = xl_ref[s] + xr_ref[d] + el_ref[j]
        z = jnp.maximum(z, 0.2 * z) * att
        matt_ref[j] = z
        return carry

    jax.lax.fori_loop(0, blk, body, 0, unroll=8)
    msum = jnp.sum(matt_ref[...], axis=2)
    t = msum.T
    p = t + pltpu.roll(t, 7, 0)
    sub = jax.lax.broadcasted_iota(jnp.int32, t.shape, 0)
    a8 = jnp.where(sub % 2 == 0, p, pltpu.roll(p, 1, 0))
    o_ref[...] = jnp.exp(a8)


def _alpha(src3, dst3, xlT, xrT, el8, att8, blk=1024):
    N = xlT.shape[0]
    E = el8.shape[0]
    return pl.pallas_call(
        functools.partial(_alpha_body, blk=blk),
        grid=(E // blk,),
        in_specs=[
            pl.BlockSpec((1, 1, blk), lambda i: (i, 0, 0),
                         memory_space=pltpu.SMEM),
            pl.BlockSpec((1, 1, blk), lambda i: (i, 0, 0),
                         memory_space=pltpu.SMEM),
            pl.BlockSpec((N, 8, 128), lambda i: (0, 0, 0)),
            pl.BlockSpec((N, 8, 128), lambda i: (0, 0, 0)),
            pl.BlockSpec((blk, 8, 128), lambda i: (i, 0, 0)),
            pl.BlockSpec((8, 128), lambda i: (0, 0)),
        ],
        out_specs=pl.BlockSpec((8, blk), lambda i: (0, i)),
        out_shape=jax.ShapeDtypeStruct((8, E), F32),
        scratch_shapes=[pltpu.VMEM((blk, 8, 128), F32)],
    )(src3, dst3, xlT, xrT, el8, att8)


def _agg_body(src_ref, dst_ref, xl_ref, ex_ref, o_ref, den_ref, *,
              n_edges, chunk=128):
    zero = jnp.zeros(o_ref.shape, F32)
    o_ref[...] = zero
    den_ref[...] = zero

    def chunk_body(c, carry):
        base = pl.multiple_of(c * chunk, chunk)
        exc = ex_ref[:, pl.ds(base, chunk)]
        for j in range(chunk):
            u, den, d_prev = carry
            e = base + j
            idx = src_ref[0, 0, e]
            d = dst_ref[0, 0, e]
            exv = jnp.broadcast_to(exc[:, j:j + 1], (8, 128))
            m = jnp.where(d == d_prev, 1.0, 0.0).astype(F32)
            u = u * m + exv * xl_ref[idx]
            den = den * m + exv
            o_ref[d] = u
            den_ref[d] = den
            carry = (u, den, d)
        return carry

    jax.lax.fori_loop(
        0, n_edges // chunk, chunk_body,
        (jnp.zeros((8, 128), F32), jnp.zeros((8, 128), F32),
         jnp.int32(-1)))
    o_ref[...] = o_ref[...] / (den_ref[...] + 1e-16)


def _agg(src3, dst3, xlT, ex8T):
    N = xlT.shape[0]
    E = ex8T.shape[1]
    return pl.pallas_call(
        functools.partial(_agg_body, n_edges=E),
        in_specs=[
            pl.BlockSpec(memory_space=pltpu.SMEM),
            pl.BlockSpec(memory_space=pltpu.SMEM),
            pl.BlockSpec((N, 8, 128), lambda: (0, 0, 0)),
            pl.BlockSpec((8, E), lambda: (0, 0)),
        ],
        out_specs=pl.BlockSpec((N, 8, 128), lambda: (0, 0, 0)),
        out_shape=jax.ShapeDtypeStruct((N, 8, 128), F32),
        scratch_shapes=[pltpu.VMEM((N, 8, 128), F32)],
    )(src3, dst3, xlT, ex8T)


def _trans_body(u_ref, bo_ref, wt_ref, bt_ref, o_ref):
    u = u_ref[...] + bo_ref[...]
    o_ref[...] = _dot(u, wt_ref[...]) + bt_ref[...]


def _trans(u, bo, wt, bt, blk=512):
    N, HC = u.shape
    DM = wt.shape[1]
    return pl.pallas_call(
        _trans_body,
        grid=(N // blk,),
        in_specs=[
            pl.BlockSpec((blk, HC), lambda i: (i, 0)),
            pl.BlockSpec((1, HC), lambda i: (0, 0)),
            pl.BlockSpec((HC, DM), lambda i: (0, 0)),
            pl.BlockSpec((1, DM), lambda i: (0, 0)),
        ],
        out_specs=pl.BlockSpec((blk, DM), lambda i: (i, 0)),
        out_shape=jax.ShapeDtypeStruct((N, DM), F32),
    )(u, bo, wt, bt)


def _ln(x, g, b):
    m = x.mean(-1, keepdims=True)
    v = ((x - m) ** 2).mean(-1, keepdims=True)
    return (x - m) / jnp.sqrt(v + 1e-5) * g + b


def _gelu(x):
    return x * 0.5 * (1.0 + jax.lax.erf(x / jnp.sqrt(2.0).astype(F32)))


def _final_body(h_ref, batch_ref, lng_ref, lnb_ref, wg_ref, bg_ref,
                wc1_ref, bc1_ref, lncg_ref, lncb_ref,
                wr1_ref, br1_ref, lnrg_ref, lnrb_ref,
                wr2_ref, br2_ref, wc2_ref, bc2_ref, o_ref, *, n_graphs):
    h = _ln(h_ref[...], lng_ref[...], lnb_ref[...])
    gate = _dot(h, wg_ref[...]) + bg_ref[...]
    N = h.shape[0]
    gi = jax.lax.broadcasted_iota(jnp.int32, (N, n_graphs), 1)
    msk = batch_ref[...] == gi
    gateb = jnp.broadcast_to(gate, (N, n_graphs))
    gm = jnp.max(jnp.where(msk, gateb, -1e30), axis=0, keepdims=True)
    gm = jnp.where(gm < -1e29, 0.0, gm)
    gw = jnp.where(msk, jnp.exp(gateb - gm), 0.0)
    den = jnp.sum(gw, axis=0, keepdims=True)
    ga = gw / (den + 1e-16)
    pooled = jnp.concatenate(
        [jnp.sum(ga[:, g:g + 1] * h, axis=0, keepdims=True)
         for g in range(n_graphs)], axis=0)
    c = _dot(pooled, wc1_ref[...]) + bc1_ref[...]
    c = _ln(c, lncg_ref[...], lncb_ref[...])
    c = _gelu(c)
    r = _dot(c, wr1_ref[...]) + br1_ref[...]
    r = _ln(r, lnrg_ref[...], lnrb_ref[...])
    r = _gelu(r)
    r = _dot(r, wr2_ref[...]) + br2_ref[...]
    c = c + r
    o_ref[...] = _dot(c, wc2_ref[...]) + bc2_ref[...]


def _final(h, batch2, p, n_graphs):
    row = lambda a: a.reshape(1, -1)
    return pl.pallas_call(
        functools.partial(_final_body, n_graphs=n_graphs),
        out_shape=jax.ShapeDtypeStruct((n_graphs, 1), F32),
    )(h, batch2,
      row(p['ln_pre_g']), row(p['ln_pre_b']), p['Wg'], row(p['bg']),
      p['Wc1'], row(p['bc1']), row(p['ln_c_g']), row(p['ln_c_b']),
      p['Wr1'], row(p['br1']), row(p['ln_r_g']), row(p['ln_r_b']),
      p['Wr2'], row(p['br2']), p['Wc2'], row(p['bc2']))


def kernel(x, edge_index, edge_attr, batch, params):
    G, S, D_IN = x.shape
    N = G * S
    E = edge_index.shape[1]

    src = edge_index[0]
    dst = edge_index[1]
    perm = jnp.argsort(dst)
    dst_s = dst[perm]
    src_s = src[perm]
    ea_s = edge_attr[perm]

    ablk = 2048
    src3 = src_s.reshape(E // ablk, 1, ablk)
    dst3 = dst_s.reshape(E // ablk, 1, ablk)
    src_flat3 = src_s.reshape(1, 1, E)
    dst_flat3 = dst_s.reshape(1, 1, E)

    row = lambda a: a.reshape(1, -1)
    h = _embed(x, params['W_in'], row(params['b_in']),
               row(params['ln_in_g']), row(params['ln_in_b']),
               params['pe'][:S])

    for lp in params['layers']:
        xl, xr = _proj(h, lp['W_l'], row(lp['b_l']), lp['W_r'], row(lp['b_r']))
        el = _eproj(ea_s, lp['W_e'])
        xlT = xl.reshape(N, 8, 128)
        xrT = xr.reshape(N, 8, 128)
        el8 = el.reshape(E, 8, 128)
        att8 = lp['att'].reshape(8, 128)
        ex8T = _alpha(src3, dst3, xlT, xrT, el8, att8, blk=ablk)
        uT = _agg(src_flat3, dst_flat3, xlT, ex8T)
        h = _trans(uT.reshape(N, 8 * 128), row(lp['b_out']), lp['Wt'],
                   row(lp['bt']))

    batch2 = batch.reshape(N, 1)
    return _final(h, batch2, params, G)
